# Initial kernel scaffold; baseline (speedup 1.0000x reference)
import jax
import jax.numpy as jnp
from jax.experimental import pallas as pl


def kernel(x, edge_index, edge_attr, batch, W1, b1, g1, be1, W2, b2, g2, be2, Wp1, bp1, Wp2, bp2):
    raise NotImplementedError("write your pallas kernel here")


# speedup vs baseline: 8.7191x; 8.7191x over previous
import functools

import jax
import jax.numpy as jnp
from jax import lax
from jax.experimental import pallas as pl
from jax.experimental.pallas import tpu as pltpu
from jax.experimental.pallas import tpu_sc as plsc

NCORES = 2
NSUB = 16
NWORK = NCORES * NSUB
CHUNK = 128
LANES = 16
BLK = 1024
EPS = 1e-5


def _sc_mesh():
    return plsc.VectorSubcoreMesh(core_axis_name="c", subcore_axis_name="s")


def _sc_degree(dst, ew, n_pad):
    ep = dst.shape[0]
    per_tile = ep // NWORK
    k_chunks = per_tile // CHUNK
    rows_per_tile = n_pad // NSUB

    @functools.partial(
        pl.kernel,
        out_type=jax.ShapeDtypeStruct((NCORES, n_pad), jnp.float32),
        mesh=_sc_mesh(),
        scratch_types=[
            pltpu.VMEM((1, CHUNK), jnp.int32),
            pltpu.VMEM((CHUNK,), jnp.float32),
            pltpu.VMEM((rows_per_tile,), jnp.float32),
            pltpu.VMEM_SHARED((n_pad,), jnp.float32),
        ],
    )
    def deg_kernel(dst_hbm, ew_hbm, out_hbm, idx_v, ew_v, z_v, acc):
        cid = lax.axis_index("c")
        sid = lax.axis_index("s")

        def zloop(i, carry):
            z_v[pl.ds(i * LANES, LANES)] = jnp.zeros((LANES,), jnp.float32)
            return carry

        lax.fori_loop(0, rows_per_tile // LANES, zloop, 0)
        pltpu.sync_copy(z_v, acc.at[pl.ds(sid * rows_per_tile, rows_per_tile)])
        plsc.subcore_barrier()

        base = (cid * NSUB + sid) * per_tile

        def body(k, carry):
            off = base + k * CHUNK
            pltpu.sync_copy(dst_hbm.at[pl.ds(off, CHUNK)], idx_v.at[0])
            pltpu.sync_copy(ew_hbm.at[pl.ds(off, CHUNK)], ew_v)
            pltpu.sync_copy(ew_v, acc.at[idx_v.at[0]], add=True)
            return carry

        lax.fori_loop(0, k_chunks, body, 0)
        plsc.subcore_barrier()
        sl = pl.ds(sid * rows_per_tile, rows_per_tile)
        pltpu.sync_copy(acc.at[sl], out_hbm.at[cid, sl])

    return deg_kernel(dst, ew)


def _sc_aggregate(src, dst, ew, y, n_pad):
    ep = src.shape[0]
    d = y.shape[1]
    per_tile = ep // NWORK
    k_chunks = per_tile // CHUNK
    rows_per_tile = n_pad // NSUB
    zrows = 40

    @functools.partial(
        pl.kernel,
        out_type=jax.ShapeDtypeStruct((NCORES, n_pad, d), jnp.float32),
        mesh=_sc_mesh(),
        scratch_types=[
            pltpu.VMEM((1, CHUNK), jnp.int32),
            pltpu.VMEM((1, CHUNK), jnp.int32),
            pltpu.VMEM((CHUNK,), jnp.float32),
            pltpu.VMEM((CHUNK, d), jnp.float32),
            pltpu.VMEM((zrows, d), jnp.float32),
            pltpu.VMEM_SHARED((n_pad, d), jnp.float32),
            pltpu.SemaphoreType.DMA,
        ],
    )
    def agg_kernel(src_hbm, dst_hbm, ew_hbm, y_hbm, out_hbm,
                   sidx, didx, ew_v, rows, zbuf, acc, sem):
        cid = lax.axis_index("c")
        sid = lax.axis_index("s")

        def zrow(i, carry):
            for j in range(d // LANES):
                zbuf[i, pl.ds(j * LANES, LANES)] = jnp.zeros((LANES,), jnp.float32)
            return carry

        lax.fori_loop(0, zrows, zrow, 0)

        def zcp(i, carry):
            pltpu.sync_copy(
                zbuf, acc.at[pl.ds(sid * rows_per_tile + i * zrows, zrows)])
            return carry

        lax.fori_loop(0, rows_per_tile // zrows, zcp, 0)
        plsc.subcore_barrier()

        base = (cid * NSUB + sid) * per_tile

        def body(k, carry):
            off = base + k * CHUNK
            pltpu.sync_copy(src_hbm.at[pl.ds(off, CHUNK)], sidx.at[0])
            pltpu.sync_copy(dst_hbm.at[pl.ds(off, CHUNK)], didx.at[0])
            pltpu.sync_copy(ew_hbm.at[pl.ds(off, CHUNK)], ew_v)
            pltpu.async_copy(y_hbm.at[sidx.at[0]], rows, sem).wait()

            def scale(g, c2):
                wv = ew_v[pl.ds(g * LANES, LANES)]
                for l in range(LANES):
                    w = wv[l]
                    r = g * LANES + l
                    for j in range(d // LANES):
                        sl = pl.ds(j * LANES, LANES)
                        rows[r, sl] = rows[r, sl] * w
                return c2

            lax.fori_loop(0, CHUNK // LANES, scale, 0)
            pltpu.sync_copy(rows, acc.at[didx.at[0]], add=True)
            return carry

        lax.fori_loop(0, k_chunks, body, 0)
        plsc.subcore_barrier()
        sl = pl.ds(sid * rows_per_tile, rows_per_tile)
        pltpu.sync_copy(acc.at[sl], out_hbm.at[cid, sl])

    return agg_kernel(src, dst, ew, y)


def _tc_prep(xp, W, parts_t):
    n_pad, d = xp.shape
    grid = n_pad // BLK

    def body(x_ref, w_ref, p_ref, y_ref, dinv_ref):
        deg = p_ref[:, 0:1] + p_ref[:, 1:2] + 1.0
        dinv = lax.rsqrt(deg)
        xw = jnp.dot(x_ref[...], w_ref[...], preferred_element_type=jnp.float32)
        y_ref[...] = xw * dinv
        dinv_ref[...] = dinv

    return pl.pallas_call(
        body,
        grid=(grid,),
        in_specs=[
            pl.BlockSpec((BLK, d), lambda i: (i, 0)),
            pl.BlockSpec((d, d), lambda i: (0, 0)),
            pl.BlockSpec((BLK, NCORES), lambda i: (i, 0)),
        ],
        out_specs=[
            pl.BlockSpec((BLK, d), lambda i: (i, 0)),
            pl.BlockSpec((BLK, 1), lambda i: (i, 0)),
        ],
        out_shape=[
            jax.ShapeDtypeStruct((n_pad, d), jnp.float32),
            jax.ShapeDtypeStruct((n_pad, 1), jnp.float32),
        ],
    )(xp, W, parts_t)


def _tc_post(parts, y, dinv2, b_row, n_real):
    n_pad, d = y.shape
    grid = n_pad // BLK

    def body(p_ref, y_ref, dinv_ref, b_ref, t_ref, s_ref):
        i = pl.program_id(0)
        t = dinv_ref[...] * (p_ref[0] + p_ref[1] + y_ref[...]) + b_ref[...]
        t_ref[...] = t
        rowid = lax.broadcasted_iota(jnp.int32, (BLK, 1), 0) + i * BLK
        tm = jnp.where(rowid < n_real, t, 0.0)

        @pl.when(i == 0)
        def _():
            s_ref[...] = jnp.zeros_like(s_ref)

        s_ref[0:1, :] += jnp.sum(tm, axis=0, keepdims=True)
        s_ref[1:2, :] += jnp.sum(tm * tm, axis=0, keepdims=True)

    return pl.pallas_call(
        body,
        grid=(grid,),
        in_specs=[
            pl.BlockSpec((NCORES, BLK, d), lambda i: (0, i, 0)),
            pl.BlockSpec((BLK, d), lambda i: (i, 0)),
            pl.BlockSpec((BLK, 1), lambda i: (i, 0)),
            pl.BlockSpec((1, d), lambda i: (0, 0)),
        ],
        out_specs=[
            pl.BlockSpec((BLK, d), lambda i: (i, 0)),
            pl.BlockSpec((2, d), lambda i: (0, 0)),
        ],
        out_shape=[
            jax.ShapeDtypeStruct((n_pad, d), jnp.float32),
            jax.ShapeDtypeStruct((2, d), jnp.float32),
        ],
    )(parts, y, dinv2, b_row)


def _tc_bn_mm(t, stats, g_row, be_row, W, dinv2, n_real):
    n_pad, d = t.shape
    grid = n_pad // BLK
    inv_n = 1.0 / n_real

    def body(t_ref, s_ref, g_ref, be_ref, w_ref, dinv_ref, y_ref):
        m = s_ref[0:1, :] * inv_n
        var = s_ref[1:2, :] * inv_n - m * m
        scale = lax.rsqrt(var + EPS) * g_ref[...]
        h = jnp.maximum((t_ref[...] - m) * scale + be_ref[...], 0.0)
        hw = jnp.dot(h, w_ref[...], preferred_element_type=jnp.float32)
        y_ref[...] = hw * dinv_ref[...]

    return pl.pallas_call(
        body,
        grid=(grid,),
        in_specs=[
            pl.BlockSpec((BLK, d), lambda i: (i, 0)),
            pl.BlockSpec((2, d), lambda i: (0, 0)),
            pl.BlockSpec((1, d), lambda i: (0, 0)),
            pl.BlockSpec((1, d), lambda i: (0, 0)),
            pl.BlockSpec((d, d), lambda i: (0, 0)),
            pl.BlockSpec((BLK, 1), lambda i: (i, 0)),
        ],
        out_specs=pl.BlockSpec((BLK, d), lambda i: (i, 0)),
        out_shape=jax.ShapeDtypeStruct((n_pad, d), jnp.float32),
    )(t, stats, g_row, be_row, W, dinv2)


def _tc_head(t, stats, g_row, be_row, batch2d, Wp1, bp1_row, Wp2, bp2_row,
             n_real, n_graphs):
    n_pad, d = t.shape
    grid = n_pad // BLK
    inv_n = 1.0 / n_real
    dh = Wp1.shape[1]

    def body(t_ref, s_ref, g_ref, be_ref, b_ref, wp1_ref, bp1_ref, wp2_ref,
             bp2_ref, out_ref, s_acc, c_acc):
        i = pl.program_id(0)
        m = s_ref[0:1, :] * inv_n
        var = s_ref[1:2, :] * inv_n - m * m
        scale = lax.rsqrt(var + EPS) * g_ref[...]
        h = jnp.maximum((t_ref[...] - m) * scale + be_ref[...], 0.0)
        gids = lax.broadcasted_iota(jnp.int32, (1, n_graphs), 1)
        oh = (b_ref[...] == gids).astype(jnp.float32)
        dn = (((0,), (0,)), ((), ()))

        @pl.when(i == 0)
        def _():
            s_acc[...] = jnp.zeros_like(s_acc)
            c_acc[...] = jnp.zeros_like(c_acc)

        s_acc[...] += lax.dot_general(oh, h, dn,
                                      preferred_element_type=jnp.float32)
        c_acc[...] += lax.dot_general(oh, jnp.ones_like(h), dn,
                                      preferred_element_type=jnp.float32)

        @pl.when(i == grid - 1)
        def _():
            pooled = s_acc[...] / jnp.maximum(c_acc[...], 1.0)
            z = jnp.dot(pooled, wp1_ref[...],
                        preferred_element_type=jnp.float32) + bp1_ref[...]
            z = jnp.maximum(z, 0.0)
            out_ref[...] = jnp.dot(z, wp2_ref[...],
                                   preferred_element_type=jnp.float32) + bp2_ref[...]

    return pl.pallas_call(
        body,
        grid=(grid,),
        in_specs=[
            pl.BlockSpec((BLK, d), lambda i: (i, 0)),
            pl.BlockSpec((2, d), lambda i: (0, 0)),
            pl.BlockSpec((1, d), lambda i: (0, 0)),
            pl.BlockSpec((1, d), lambda i: (0, 0)),
            pl.BlockSpec((BLK, 1), lambda i: (i, 0)),
            pl.BlockSpec((d, dh), lambda i: (0, 0)),
            pl.BlockSpec((1, dh), lambda i: (0, 0)),
            pl.BlockSpec((dh, 1), lambda i: (0, 0)),
            pl.BlockSpec((1, 1), lambda i: (0, 0)),
        ],
        out_specs=pl.BlockSpec((n_graphs, 1), lambda i: (0, 0)),
        out_shape=jax.ShapeDtypeStruct((n_graphs, 1), jnp.float32),
        scratch_shapes=[
            pltpu.VMEM((n_graphs, d), jnp.float32),
            pltpu.VMEM((n_graphs, d), jnp.float32),
        ],
    )(t, stats, g_row, be_row, batch2d, Wp1, bp1_row, Wp2, bp2_row)


def kernel(x, edge_index, edge_attr, batch,
           W1, b1, g1, be1, W2, b2, g2, be2, Wp1, bp1, Wp2, bp2):
    n, d = x.shape
    e = edge_index.shape[1]
    n_graphs = 64

    n_pad = ((n + BLK - 1) // BLK) * BLK
    estep = NWORK * CHUNK
    ep = ((e + estep - 1) // estep) * estep

    src = jnp.concatenate(
        [edge_index[0], jnp.zeros((ep - e,), jnp.int32)])
    dst = jnp.concatenate(
        [edge_index[1], jnp.full((ep - e,), n, jnp.int32)])
    ew = jnp.concatenate(
        [edge_attr[:, 0], jnp.zeros((ep - e,), jnp.float32)])

    xp = jnp.pad(x, ((0, n_pad - n), (0, 0)))
    batch2d = jnp.pad(batch, (0, n_pad - n),
                      constant_values=n_graphs).reshape(n_pad, 1)

    b1r, g1r, be1r = b1.reshape(1, d), g1.reshape(1, d), be1.reshape(1, d)
    b2r, g2r, be2r = b2.reshape(1, d), g2.reshape(1, d), be2.reshape(1, d)
    bp1r = bp1.reshape(1, -1)
    bp2r = bp2.reshape(1, 1)

    deg_parts = _sc_degree(dst, ew, n_pad)
    parts_t = deg_parts.T

    y1, dinv2 = _tc_prep(xp, W1, parts_t)
    p1 = _sc_aggregate(src, dst, ew, y1, n_pad)
    t1, s1 = _tc_post(p1, y1, dinv2, b1r, n)
    y2 = _tc_bn_mm(t1, s1, g1r, be1r, W2, dinv2, n)
    p2 = _sc_aggregate(src, dst, ew, y2, n_pad)
    t2, s2 = _tc_post(p2, y2, dinv2, b2r, n)
    out = _tc_head(t2, s2, g2r, be2r, batch2d, Wp1, bp1r, Wp2, bp2r,
                   n, n_graphs)
    return out


# speedup vs baseline: 9.2264x; 1.0582x over previous
import functools

import jax
import jax.numpy as jnp
from jax import lax
from jax.experimental import pallas as pl
from jax.experimental.pallas import tpu as pltpu
from jax.experimental.pallas import tpu_sc as plsc

NCORES = 2
NSUB = 16
NWORK = NCORES * NSUB
CHUNK = 128
LANES = 16
BLK = 1024
EPS = 1e-5


def _sc_mesh():
    return plsc.VectorSubcoreMesh(core_axis_name="c", subcore_axis_name="s")


def _sc_degree(dst2, ew2, n_pad):
    k_chunks = dst2.shape[0] // NWORK
    rows_per_tile = n_pad // NSUB
    fire = 8

    @functools.partial(
        pl.kernel,
        out_type=jax.ShapeDtypeStruct((NCORES, n_pad), jnp.float32),
        mesh=_sc_mesh(),
        scratch_types=[
            pltpu.VMEM((k_chunks, CHUNK), jnp.int32),
            pltpu.VMEM((k_chunks, CHUNK), jnp.float32),
            pltpu.VMEM((rows_per_tile,), jnp.float32),
            pltpu.VMEM_SHARED((n_pad,), jnp.float32),
            pltpu.SemaphoreType.DMA,
        ],
    )
    def deg_kernel(dst_hbm, ew_hbm, out_hbm, idx_v, ew_v, z_v, acc, sem):
        cid = lax.axis_index("c")
        sid = lax.axis_index("s")

        def zloop(i, carry):
            z_v[pl.ds(i * LANES, LANES)] = jnp.zeros((LANES,), jnp.float32)
            return carry

        lax.fori_loop(0, rows_per_tile // LANES, zloop, 0)
        pltpu.sync_copy(z_v, acc.at[pl.ds(sid * rows_per_tile, rows_per_tile)])

        base = (cid * NSUB + sid) * k_chunks
        pltpu.sync_copy(dst_hbm.at[pl.ds(base, k_chunks)], idx_v)
        pltpu.sync_copy(ew_hbm.at[pl.ds(base, k_chunks)], ew_v)
        plsc.subcore_barrier()

        def body(g, carry):
            descs = []
            for u in range(fire):
                k = g * fire + u
                descs.append(pltpu.async_copy(
                    ew_v.at[k], acc.at[idx_v.at[k]], sem, add=True))
            for dsc in descs:
                dsc.wait()
            return carry

        lax.fori_loop(0, k_chunks // fire, body, 0)
        plsc.subcore_barrier()
        sl = pl.ds(sid * rows_per_tile, rows_per_tile)
        pltpu.sync_copy(acc.at[sl], out_hbm.at[cid, sl])

    return deg_kernel(dst2, ew2)


def _sc_aggregate(src2, dst2, ew2, y, n_pad):
    d = y.shape[1]
    k_chunks = src2.shape[0] // NWORK
    npairs = k_chunks // 2
    rows_per_tile = n_pad // NSUB
    zrows = 16

    @functools.partial(
        pl.kernel,
        out_type=jax.ShapeDtypeStruct((NCORES, n_pad, d), jnp.float32),
        mesh=_sc_mesh(),
        scratch_types=[
            pltpu.VMEM((2, CHUNK), jnp.int32),
            pltpu.VMEM((2, CHUNK), jnp.int32),
            pltpu.VMEM((2, CHUNK), jnp.int32),
            pltpu.VMEM((2, CHUNK), jnp.float32),
            pltpu.VMEM((CHUNK, d), jnp.float32),
            pltpu.VMEM((CHUNK, d), jnp.float32),
            pltpu.VMEM((zrows, d), jnp.float32),
            pltpu.VMEM_SHARED((n_pad, d), jnp.float32),
            pltpu.SemaphoreType.DMA,
            pltpu.SemaphoreType.DMA,
            pltpu.SemaphoreType.DMA,
            pltpu.SemaphoreType.DMA,
            pltpu.SemaphoreType.DMA,
            pltpu.SemaphoreType.DMA,
            pltpu.SemaphoreType.DMA,
        ],
    )
    def agg_kernel(src_hbm, dst_hbm, ew_hbm, y_hbm, out_hbm,
                   sidx, didx, didx_s, ew_v, rows0, rows1, zbuf, acc,
                   isem0, isem1, gsem0, gsem1, ssem0, ssem1, zsem):
        cid = lax.axis_index("c")
        sid = lax.axis_index("s")

        def zrow(i, carry):
            for j in range(d // LANES):
                zbuf[i, pl.ds(j * LANES, LANES)] = jnp.zeros((LANES,), jnp.float32)
            return carry

        lax.fori_loop(0, zrows, zrow, 0)

        for i in range(rows_per_tile // zrows):
            pltpu.async_copy(
                zbuf, acc.at[pl.ds(sid * rows_per_tile + i * zrows, zrows)],
                zsem)
        base = (cid * NSUB + sid) * k_chunks

        def fetch_idx(k, b, isem):
            pltpu.async_copy(src_hbm.at[k + base], sidx.at[b], isem)
            pltpu.async_copy(dst_hbm.at[k + base], didx.at[b], isem)
            pltpu.async_copy(ew_hbm.at[k + base], ew_v.at[b], isem)

        def wait_idx(b, isem):
            pltpu.make_async_copy(src_hbm.at[0], sidx.at[b], isem).wait()
            pltpu.make_async_copy(dst_hbm.at[0], didx.at[b], isem).wait()
            pltpu.make_async_copy(ew_hbm.at[0], ew_v.at[b], isem).wait()

        def scale(rows, b):
            def grp(g, c2):
                wv = ew_v[b, pl.ds(g * LANES, LANES)]
                for l in range(LANES):
                    w = wv[l]
                    r = g * LANES + l
                    for j in range(d // LANES):
                        sl = pl.ds(j * LANES, LANES)
                        rows[r, sl] = rows[r, sl] * w
                return c2

            lax.fori_loop(0, CHUNK // LANES, grp, 0)

        def gather(b, rows, gsem):
            return pltpu.async_copy(y_hbm.at[sidx.at[b]], rows, gsem)

        def wait_gather(b, rows, gsem):
            pltpu.make_async_copy(y_hbm.at[sidx.at[b]], rows, gsem).wait()

        def scatter(b, rows, ssem):
            for j in range(CHUNK // LANES):
                sl = pl.ds(j * LANES, LANES)
                didx_s[b, sl] = didx[b, sl]
            return pltpu.async_copy(rows, acc.at[didx_s.at[b]], ssem, add=True)

        def wait_scatter(b, rows, ssem):
            pltpu.make_async_copy(rows, acc.at[didx_s.at[b]], ssem).wait()

        for i in range(rows_per_tile // zrows):
            pltpu.make_async_copy(
                zbuf, acc.at[pl.ds(sid * rows_per_tile + i * zrows, zrows)],
                zsem).wait()
        plsc.subcore_barrier()

        fetch_idx(0, 0, isem0)
        wait_idx(0, isem0)
        gather(0, rows0, gsem0)
        fetch_idx(1, 1, isem1)

        def body(j, carry):
            a = j * 2

            @pl.when(j > 0)
            def _():
                wait_scatter(1, rows1, ssem1)

            wait_idx(1, isem1)
            gather(1, rows1, gsem1)
            wait_gather(0, rows0, gsem0)
            scale(rows0, 0)
            scatter(0, rows0, ssem0)
            wait_gather(1, rows1, gsem1)
            scale(rows1, 1)
            wait_scatter(0, rows0, ssem0)

            @pl.when(j < npairs - 1)
            def _():
                fetch_idx(a + 2, 0, isem0)

            scatter(1, rows1, ssem1)

            @pl.when(j < npairs - 1)
            def _():
                wait_idx(0, isem0)
                gather(0, rows0, gsem0)
                fetch_idx(a + 3, 1, isem1)

            return carry

        lax.fori_loop(0, npairs, body, 0)
        wait_scatter(1, rows1, ssem1)
        plsc.subcore_barrier()
        sl = pl.ds(sid * rows_per_tile, rows_per_tile)
        pltpu.sync_copy(acc.at[sl], out_hbm.at[cid, sl])

    return agg_kernel(src2, dst2, ew2, y)


def _tc_prep(xp, W, parts_t):
    n_pad, d = xp.shape
    grid = n_pad // BLK

    def body(x_ref, w_ref, p_ref, y_ref, dinv_ref):
        deg = p_ref[:, 0:1] + p_ref[:, 1:2] + 1.0
        dinv = lax.rsqrt(deg)
        xw = jnp.dot(x_ref[...], w_ref[...], preferred_element_type=jnp.float32)
        y_ref[...] = xw * dinv
        dinv_ref[...] = dinv

    return pl.pallas_call(
        body,
        grid=(grid,),
        in_specs=[
            pl.BlockSpec((BLK, d), lambda i: (i, 0)),
            pl.BlockSpec((d, d), lambda i: (0, 0)),
            pl.BlockSpec((BLK, NCORES), lambda i: (i, 0)),
        ],
        out_specs=[
            pl.BlockSpec((BLK, d), lambda i: (i, 0)),
            pl.BlockSpec((BLK, 1), lambda i: (i, 0)),
        ],
        out_shape=[
            jax.ShapeDtypeStruct((n_pad, d), jnp.float32),
            jax.ShapeDtypeStruct((n_pad, 1), jnp.float32),
        ],
    )(xp, W, parts_t)


def _tc_post(parts, y, dinv2, b_row, n_real):
    n_pad, d = y.shape
    grid = n_pad // BLK

    def body(p_ref, y_ref, dinv_ref, b_ref, t_ref, s_ref):
        i = pl.program_id(0)
        t = dinv_ref[...] * (p_ref[0] + p_ref[1] + y_ref[...]) + b_ref[...]
        t_ref[...] = t
        rowid = lax.broadcasted_iota(jnp.int32, (BLK, 1), 0) + i * BLK
        tm = jnp.where(rowid < n_real, t, 0.0)

        @pl.when(i == 0)
        def _():
            s_ref[...] = jnp.zeros_like(s_ref)

        s_ref[0:1, :] += jnp.sum(tm, axis=0, keepdims=True)
        s_ref[1:2, :] += jnp.sum(tm * tm, axis=0, keepdims=True)

    return pl.pallas_call(
        body,
        grid=(grid,),
        in_specs=[
            pl.BlockSpec((NCORES, BLK, d), lambda i: (0, i, 0)),
            pl.BlockSpec((BLK, d), lambda i: (i, 0)),
            pl.BlockSpec((BLK, 1), lambda i: (i, 0)),
            pl.BlockSpec((1, d), lambda i: (0, 0)),
        ],
        out_specs=[
            pl.BlockSpec((BLK, d), lambda i: (i, 0)),
            pl.BlockSpec((2, d), lambda i: (0, 0)),
        ],
        out_shape=[
            jax.ShapeDtypeStruct((n_pad, d), jnp.float32),
            jax.ShapeDtypeStruct((2, d), jnp.float32),
        ],
    )(parts, y, dinv2, b_row)


def _tc_bn_mm(t, stats, g_row, be_row, W, dinv2, n_real):
    n_pad, d = t.shape
    grid = n_pad // BLK
    inv_n = 1.0 / n_real

    def body(t_ref, s_ref, g_ref, be_ref, w_ref, dinv_ref, y_ref):
        m = s_ref[0:1, :] * inv_n
        var = s_ref[1:2, :] * inv_n - m * m
        scale = lax.rsqrt(var + EPS) * g_ref[...]
        h = jnp.maximum((t_ref[...] - m) * scale + be_ref[...], 0.0)
        hw = jnp.dot(h, w_ref[...], preferred_element_type=jnp.float32)
        y_ref[...] = hw * dinv_ref[...]

    return pl.pallas_call(
        body,
        grid=(grid,),
        in_specs=[
            pl.BlockSpec((BLK, d), lambda i: (i, 0)),
            pl.BlockSpec((2, d), lambda i: (0, 0)),
            pl.BlockSpec((1, d), lambda i: (0, 0)),
            pl.BlockSpec((1, d), lambda i: (0, 0)),
            pl.BlockSpec((d, d), lambda i: (0, 0)),
            pl.BlockSpec((BLK, 1), lambda i: (i, 0)),
        ],
        out_specs=pl.BlockSpec((BLK, d), lambda i: (i, 0)),
        out_shape=jax.ShapeDtypeStruct((n_pad, d), jnp.float32),
    )(t, stats, g_row, be_row, W, dinv2)


def _tc_head(t, stats, g_row, be_row, batch2d, Wp1, bp1_row, Wp2, bp2_row,
             n_real, n_graphs):
    n_pad, d = t.shape
    grid = n_pad // BLK
    inv_n = 1.0 / n_real
    dh = Wp1.shape[1]

    def body(t_ref, s_ref, g_ref, be_ref, b_ref, wp1_ref, bp1_ref, wp2_ref,
             bp2_ref, out_ref, s_acc, c_acc):
        i = pl.program_id(0)
        m = s_ref[0:1, :] * inv_n
        var = s_ref[1:2, :] * inv_n - m * m
        scale = lax.rsqrt(var + EPS) * g_ref[...]
        h = jnp.maximum((t_ref[...] - m) * scale + be_ref[...], 0.0)
        gids = lax.broadcasted_iota(jnp.int32, (1, n_graphs), 1)
        oh = (b_ref[...] == gids).astype(jnp.float32)
        dn = (((0,), (0,)), ((), ()))

        @pl.when(i == 0)
        def _():
            s_acc[...] = jnp.zeros_like(s_acc)
            c_acc[...] = jnp.zeros_like(c_acc)

        s_acc[...] += lax.dot_general(oh, h, dn,
                                      preferred_element_type=jnp.float32)
        c_acc[...] += lax.dot_general(oh, jnp.ones_like(h), dn,
                                      preferred_element_type=jnp.float32)

        @pl.when(i == grid - 1)
        def _():
            pooled = s_acc[...] / jnp.maximum(c_acc[...], 1.0)
            z = jnp.dot(pooled, wp1_ref[...],
                        preferred_element_type=jnp.float32) + bp1_ref[...]
            z = jnp.maximum(z, 0.0)
            out_ref[...] = jnp.dot(z, wp2_ref[...],
                                   preferred_element_type=jnp.float32) + bp2_ref[...]

    return pl.pallas_call(
        body,
        grid=(grid,),
        in_specs=[
            pl.BlockSpec((BLK, d), lambda i: (i, 0)),
            pl.BlockSpec((2, d), lambda i: (0, 0)),
            pl.BlockSpec((1, d), lambda i: (0, 0)),
            pl.BlockSpec((1, d), lambda i: (0, 0)),
            pl.BlockSpec((BLK, 1), lambda i: (i, 0)),
            pl.BlockSpec((d, dh), lambda i: (0, 0)),
            pl.BlockSpec((1, dh), lambda i: (0, 0)),
            pl.BlockSpec((dh, 1), lambda i: (0, 0)),
            pl.BlockSpec((1, 1), lambda i: (0, 0)),
        ],
        out_specs=pl.BlockSpec((n_graphs, 1), lambda i: (0, 0)),
        out_shape=jax.ShapeDtypeStruct((n_graphs, 1), jnp.float32),
        scratch_shapes=[
            pltpu.VMEM((n_graphs, d), jnp.float32),
            pltpu.VMEM((n_graphs, d), jnp.float32),
        ],
    )(t, stats, g_row, be_row, batch2d, Wp1, bp1_row, Wp2, bp2_row)


def kernel(x, edge_index, edge_attr, batch,
           W1, b1, g1, be1, W2, b2, g2, be2, Wp1, bp1, Wp2, bp2):
    n, d = x.shape
    e = edge_index.shape[1]
    n_graphs = 64

    n_pad = ((n + BLK - 1) // BLK) * BLK
    estep = NWORK * CHUNK * 2
    ep = ((e + estep - 1) // estep) * estep

    src = jnp.concatenate(
        [edge_index[0], jnp.zeros((ep - e,), jnp.int32)])
    dst = jnp.concatenate(
        [edge_index[1], jnp.full((ep - e,), n, jnp.int32)])
    ew = jnp.concatenate(
        [edge_attr[:, 0], jnp.zeros((ep - e,), jnp.float32)])
    src = src.reshape(ep // CHUNK, CHUNK)
    dst = dst.reshape(ep // CHUNK, CHUNK)
    ew = ew.reshape(ep // CHUNK, CHUNK)

    xp = jnp.pad(x, ((0, n_pad - n), (0, 0)))
    batch2d = jnp.pad(batch, (0, n_pad - n),
                      constant_values=n_graphs).reshape(n_pad, 1)

    b1r, g1r, be1r = b1.reshape(1, d), g1.reshape(1, d), be1.reshape(1, d)
    b2r, g2r, be2r = b2.reshape(1, d), g2.reshape(1, d), be2.reshape(1, d)
    bp1r = bp1.reshape(1, -1)
    bp2r = bp2.reshape(1, 1)

    deg_parts = _sc_degree(dst, ew, n_pad)
    parts_t = deg_parts.T

    y1, dinv2 = _tc_prep(xp, W1, parts_t)
    p1 = _sc_aggregate(src, dst, ew, y1, n_pad)
    t1, s1 = _tc_post(p1, y1, dinv2, b1r, n)
    y2 = _tc_bn_mm(t1, s1, g1r, be1r, W2, dinv2, n)
    p2 = _sc_aggregate(src, dst, ew, y2, n_pad)
    t2, s2 = _tc_post(p2, y2, dinv2, b2r, n)
    out = _tc_head(t2, s2, g2r, be2r, batch2d, Wp1, bp1r, Wp2, bp2r,
                   n, n_graphs)
    return out


# speedup vs baseline: 23.7784x; 2.5772x over previous
import functools

import jax
import jax.numpy as jnp
from jax import lax
from jax.experimental import pallas as pl
from jax.experimental.pallas import tpu as pltpu
from jax.experimental.pallas import tpu_sc as plsc

NCORES = 2
NSUB = 16
NWORK = NCORES * NSUB
CHUNK = 128
LANES = 16
BLK = 1024
EPS = 1e-5


def _sc_mesh():
    return plsc.VectorSubcoreMesh(core_axis_name="c", subcore_axis_name="s")


def _sc_degree(dst2, ew2, n_pad):
    k_chunks = dst2.shape[0] // NWORK
    rows_per_tile = n_pad // NSUB
    fire = 8

    @functools.partial(
        pl.kernel,
        out_type=jax.ShapeDtypeStruct((NCORES, n_pad), jnp.float32),
        mesh=_sc_mesh(),
        scratch_types=[
            pltpu.VMEM((k_chunks, CHUNK), jnp.int32),
            pltpu.VMEM((k_chunks, CHUNK), jnp.float32),
            pltpu.VMEM((rows_per_tile,), jnp.float32),
            pltpu.VMEM_SHARED((n_pad,), jnp.float32),
            pltpu.SemaphoreType.DMA,
        ],
    )
    def deg_kernel(dst_hbm, ew_hbm, out_hbm, idx_v, ew_v, z_v, acc, sem):
        cid = lax.axis_index("c")
        sid = lax.axis_index("s")

        def zloop(i, carry):
            z_v[pl.ds(i * LANES, LANES)] = jnp.zeros((LANES,), jnp.float32)
            return carry

        lax.fori_loop(0, rows_per_tile // LANES, zloop, 0)
        pltpu.sync_copy(z_v, acc.at[pl.ds(sid * rows_per_tile, rows_per_tile)])

        base = (cid * NSUB + sid) * k_chunks
        pltpu.sync_copy(dst_hbm.at[pl.ds(base, k_chunks)], idx_v)
        pltpu.sync_copy(ew_hbm.at[pl.ds(base, k_chunks)], ew_v)
        plsc.subcore_barrier()

        def body(g, carry):
            descs = []
            for u in range(fire):
                k = g * fire + u
                descs.append(pltpu.async_copy(
                    ew_v.at[k], acc.at[idx_v.at[k]], sem, add=True))
            for dsc in descs:
                dsc.wait()
            return carry

        lax.fori_loop(0, k_chunks // fire, body, 0)
        plsc.subcore_barrier()
        sl = pl.ds(sid * rows_per_tile, rows_per_tile)
        pltpu.sync_copy(acc.at[sl], out_hbm.at[cid, sl])

    return deg_kernel(dst2, ew2)


def _sc_aggregate(src2, dst2, ew2, y, n_pad):
    d = y.shape[1]
    k_chunks = src2.shape[0] // NWORK
    npairs = k_chunks // 2
    rows_per_tile = n_pad // NSUB
    zrows = 16

    @functools.partial(
        pl.kernel,
        out_type=jax.ShapeDtypeStruct((NCORES, n_pad, d), jnp.float32),
        mesh=_sc_mesh(),
        scratch_types=[
            pltpu.VMEM((2, CHUNK), jnp.int32),
            pltpu.VMEM((2, CHUNK), jnp.int32),
            pltpu.VMEM((2, CHUNK), jnp.int32),
            pltpu.VMEM((2, CHUNK), jnp.float32),
            pltpu.VMEM((CHUNK, d), jnp.float32),
            pltpu.VMEM((CHUNK, d), jnp.float32),
            pltpu.VMEM((zrows, d), jnp.float32),
            pltpu.VMEM_SHARED((n_pad, d), jnp.float32),
            pltpu.SemaphoreType.DMA,
            pltpu.SemaphoreType.DMA,
            pltpu.SemaphoreType.DMA,
            pltpu.SemaphoreType.DMA,
            pltpu.SemaphoreType.DMA,
            pltpu.SemaphoreType.DMA,
            pltpu.SemaphoreType.DMA,
        ],
    )
    def agg_kernel(src_hbm, dst_hbm, ew_hbm, y_hbm, out_hbm,
                   sidx, didx, didx_s, ew_v, rows0, rows1, zbuf, acc,
                   isem0, isem1, gsem0, gsem1, ssem0, ssem1, zsem):
        cid = lax.axis_index("c")
        sid = lax.axis_index("s")

        def zrow(i, carry):
            for j in range(d // LANES):
                zbuf[i, pl.ds(j * LANES, LANES)] = jnp.zeros((LANES,), jnp.float32)
            return carry

        lax.fori_loop(0, zrows, zrow, 0)

        for i in range(rows_per_tile // zrows):
            pltpu.async_copy(
                zbuf, acc.at[pl.ds(sid * rows_per_tile + i * zrows, zrows)],
                zsem)
        base = (cid * NSUB + sid) * k_chunks

        def fetch_idx(k, b, isem):
            pltpu.async_copy(src_hbm.at[k + base], sidx.at[b], isem)
            pltpu.async_copy(dst_hbm.at[k + base], didx.at[b], isem)
            pltpu.async_copy(ew_hbm.at[k + base], ew_v.at[b], isem)

        def wait_idx(b, isem):
            pltpu.make_async_copy(src_hbm.at[0], sidx.at[b], isem).wait()
            pltpu.make_async_copy(dst_hbm.at[0], didx.at[b], isem).wait()
            pltpu.make_async_copy(ew_hbm.at[0], ew_v.at[b], isem).wait()

        def scale(rows, b):
            def grp(g, c2):
                wv = ew_v[b, pl.ds(g * LANES, LANES)]
                for l in range(LANES):
                    w = wv[l]
                    r = g * LANES + l
                    for j in range(d // LANES):
                        sl = pl.ds(j * LANES, LANES)
                        rows[r, sl] = rows[r, sl] * w
                return c2

            lax.fori_loop(0, CHUNK // LANES, grp, 0)

        def gather(b, rows, gsem):
            return pltpu.async_copy(y_hbm.at[sidx.at[b]], rows, gsem)

        def wait_gather(b, rows, gsem):
            pltpu.make_async_copy(y_hbm.at[sidx.at[b]], rows, gsem).wait()

        def scatter(b, rows, ssem):
            for j in range(CHUNK // LANES):
                sl = pl.ds(j * LANES, LANES)
                didx_s[b, sl] = didx[b, sl]
            return pltpu.async_copy(rows, acc.at[didx_s.at[b]], ssem, add=True)

        def wait_scatter(b, rows, ssem):
            pltpu.make_async_copy(rows, acc.at[didx_s.at[b]], ssem).wait()

        for i in range(rows_per_tile // zrows):
            pltpu.make_async_copy(
                zbuf, acc.at[pl.ds(sid * rows_per_tile + i * zrows, zrows)],
                zsem).wait()
        plsc.subcore_barrier()

        fetch_idx(0, 0, isem0)
        wait_idx(0, isem0)
        gather(0, rows0, gsem0)
        fetch_idx(1, 1, isem1)

        def body(j, carry):
            a = j * 2

            @pl.when(j > 0)
            def _():
                wait_scatter(1, rows1, ssem1)

            wait_idx(1, isem1)
            gather(1, rows1, gsem1)
            wait_gather(0, rows0, gsem0)
            scale(rows0, 0)
            scatter(0, rows0, ssem0)
            wait_gather(1, rows1, gsem1)
            scale(rows1, 1)
            wait_scatter(0, rows0, ssem0)

            @pl.when(j < npairs - 1)
            def _():
                fetch_idx(a + 2, 0, isem0)

            scatter(1, rows1, ssem1)

            @pl.when(j < npairs - 1)
            def _():
                wait_idx(0, isem0)
                gather(0, rows0, gsem0)
                fetch_idx(a + 3, 1, isem1)

            return carry

        lax.fori_loop(0, npairs, body, 0)
        wait_scatter(1, rows1, ssem1)
        plsc.subcore_barrier()
        sl = pl.ds(sid * rows_per_tile, rows_per_tile)
        pltpu.sync_copy(acc.at[sl], out_hbm.at[cid, sl])

    return agg_kernel(src2, dst2, ew2, y)


def _tc_prep(xp, W, parts_t):
    n_pad, d = xp.shape
    grid = n_pad // BLK

    def body(x_ref, w_ref, p_ref, y_ref, dinv_ref):
        deg = p_ref[:, 0:1] + p_ref[:, 1:2] + 1.0
        dinv = lax.rsqrt(deg)
        xw = jnp.dot(x_ref[...], w_ref[...], preferred_element_type=jnp.float32)
        y_ref[...] = xw * dinv
        dinv_ref[...] = dinv

    return pl.pallas_call(
        body,
        grid=(grid,),
        in_specs=[
            pl.BlockSpec((BLK, d), lambda i: (i, 0)),
            pl.BlockSpec((d, d), lambda i: (0, 0)),
            pl.BlockSpec((BLK, NCORES), lambda i: (i, 0)),
        ],
        out_specs=[
            pl.BlockSpec((BLK, d), lambda i: (i, 0)),
            pl.BlockSpec((BLK, 1), lambda i: (i, 0)),
        ],
        out_shape=[
            jax.ShapeDtypeStruct((n_pad, d), jnp.float32),
            jax.ShapeDtypeStruct((n_pad, 1), jnp.float32),
        ],
    )(xp, W, parts_t)


def _tc_post(parts, y, dinv2, b_row, n_real):
    n_pad, d = y.shape
    grid = n_pad // BLK

    def body(p_ref, y_ref, dinv_ref, b_ref, t_ref, s_ref):
        i = pl.program_id(0)
        t = dinv_ref[...] * (p_ref[0] + p_ref[1] + y_ref[...]) + b_ref[...]
        t_ref[...] = t
        rowid = lax.broadcasted_iota(jnp.int32, (BLK, 1), 0) + i * BLK
        tm = jnp.where(rowid < n_real, t, 0.0)

        @pl.when(i == 0)
        def _():
            s_ref[...] = jnp.zeros_like(s_ref)

        s_ref[0:1, :] += jnp.sum(tm, axis=0, keepdims=True)
        s_ref[1:2, :] += jnp.sum(tm * tm, axis=0, keepdims=True)

    return pl.pallas_call(
        body,
        grid=(grid,),
        in_specs=[
            pl.BlockSpec((NCORES, BLK, d), lambda i: (0, i, 0)),
            pl.BlockSpec((BLK, d), lambda i: (i, 0)),
            pl.BlockSpec((BLK, 1), lambda i: (i, 0)),
            pl.BlockSpec((1, d), lambda i: (0, 0)),
        ],
        out_specs=[
            pl.BlockSpec((BLK, d), lambda i: (i, 0)),
            pl.BlockSpec((2, d), lambda i: (0, 0)),
        ],
        out_shape=[
            jax.ShapeDtypeStruct((n_pad, d), jnp.float32),
            jax.ShapeDtypeStruct((2, d), jnp.float32),
        ],
    )(parts, y, dinv2, b_row)


def _tc_bn_mm(t, stats, g_row, be_row, W, dinv2, n_real):
    n_pad, d = t.shape
    grid = n_pad // BLK
    inv_n = 1.0 / n_real

    def body(t_ref, s_ref, g_ref, be_ref, w_ref, dinv_ref, y_ref):
        m = s_ref[0:1, :] * inv_n
        var = s_ref[1:2, :] * inv_n - m * m
        scale = lax.rsqrt(var + EPS) * g_ref[...]
        h = jnp.maximum((t_ref[...] - m) * scale + be_ref[...], 0.0)
        hw = jnp.dot(h, w_ref[...], preferred_element_type=jnp.float32)
        y_ref[...] = hw * dinv_ref[...]

    return pl.pallas_call(
        body,
        grid=(grid,),
        in_specs=[
            pl.BlockSpec((BLK, d), lambda i: (i, 0)),
            pl.BlockSpec((2, d), lambda i: (0, 0)),
            pl.BlockSpec((1, d), lambda i: (0, 0)),
            pl.BlockSpec((1, d), lambda i: (0, 0)),
            pl.BlockSpec((d, d), lambda i: (0, 0)),
            pl.BlockSpec((BLK, 1), lambda i: (i, 0)),
        ],
        out_specs=pl.BlockSpec((BLK, d), lambda i: (i, 0)),
        out_shape=jax.ShapeDtypeStruct((n_pad, d), jnp.float32),
    )(t, stats, g_row, be_row, W, dinv2)


def _tc_head(t, stats, g_row, be_row, batch2d, Wp1, bp1_row, Wp2, bp2_row,
             n_real, n_graphs):
    n_pad, d = t.shape
    grid = n_pad // BLK
    inv_n = 1.0 / n_real
    dh = Wp1.shape[1]

    def body(t_ref, s_ref, g_ref, be_ref, b_ref, wp1_ref, bp1_ref, wp2_ref,
             bp2_ref, out_ref, s_acc, c_acc):
        i = pl.program_id(0)
        m = s_ref[0:1, :] * inv_n
        var = s_ref[1:2, :] * inv_n - m * m
        scale = lax.rsqrt(var + EPS) * g_ref[...]
        h = jnp.maximum((t_ref[...] - m) * scale + be_ref[...], 0.0)
        gids = lax.broadcasted_iota(jnp.int32, (1, n_graphs), 1)
        oh = (b_ref[...] == gids).astype(jnp.float32)
        dn = (((0,), (0,)), ((), ()))

        @pl.when(i == 0)
        def _():
            s_acc[...] = jnp.zeros_like(s_acc)
            c_acc[...] = jnp.zeros_like(c_acc)

        s_acc[...] += lax.dot_general(oh, h, dn,
                                      preferred_element_type=jnp.float32)
        c_acc[...] += lax.dot_general(oh, jnp.ones_like(h), dn,
                                      preferred_element_type=jnp.float32)

        @pl.when(i == grid - 1)
        def _():
            pooled = s_acc[...] / jnp.maximum(c_acc[...], 1.0)
            z = jnp.dot(pooled, wp1_ref[...],
                        preferred_element_type=jnp.float32) + bp1_ref[...]
            z = jnp.maximum(z, 0.0)
            out_ref[...] = jnp.dot(z, wp2_ref[...],
                                   preferred_element_type=jnp.float32) + bp2_ref[...]

    return pl.pallas_call(
        body,
        grid=(grid,),
        in_specs=[
            pl.BlockSpec((BLK, d), lambda i: (i, 0)),
            pl.BlockSpec((2, d), lambda i: (0, 0)),
            pl.BlockSpec((1, d), lambda i: (0, 0)),
            pl.BlockSpec((1, d), lambda i: (0, 0)),
            pl.BlockSpec((BLK, 1), lambda i: (i, 0)),
            pl.BlockSpec((d, dh), lambda i: (0, 0)),
            pl.BlockSpec((1, dh), lambda i: (0, 0)),
            pl.BlockSpec((dh, 1), lambda i: (0, 0)),
            pl.BlockSpec((1, 1), lambda i: (0, 0)),
        ],
        out_specs=pl.BlockSpec((n_graphs, 1), lambda i: (0, 0)),
        out_shape=jax.ShapeDtypeStruct((n_graphs, 1), jnp.float32),
        scratch_shapes=[
            pltpu.VMEM((n_graphs, d), jnp.float32),
            pltpu.VMEM((n_graphs, d), jnp.float32),
        ],
    )(t, stats, g_row, be_row, batch2d, Wp1, bp1_row, Wp2, bp2_row)


def kernel(x, edge_index, edge_attr, batch,
           W1, b1, g1, be1, W2, b2, g2, be2, Wp1, bp1, Wp2, bp2):
    n, d = x.shape
    e = edge_index.shape[1]
    n_graphs = 64

    n_pad = ((n + BLK - 1) // BLK) * BLK
    estep = NWORK * CHUNK * 2
    ep = ((e + estep - 1) // estep) * estep

    pad_ar = jnp.arange(ep - e, dtype=jnp.int32)
    src = jnp.concatenate([edge_index[0], pad_ar % n])
    dst = jnp.concatenate([edge_index[1], n + pad_ar % (n_pad - n)])
    ew = jnp.concatenate(
        [edge_attr[:, 0], jnp.zeros((ep - e,), jnp.float32)])
    src = src.reshape(ep // CHUNK, CHUNK)
    dst = dst.reshape(ep // CHUNK, CHUNK)
    ew = ew.reshape(ep // CHUNK, CHUNK)

    xp = jnp.pad(x, ((0, n_pad - n), (0, 0)))
    batch2d = jnp.pad(batch, (0, n_pad - n),
                      constant_values=n_graphs).reshape(n_pad, 1)

    b1r, g1r, be1r = b1.reshape(1, d), g1.reshape(1, d), be1.reshape(1, d)
    b2r, g2r, be2r = b2.reshape(1, d), g2.reshape(1, d), be2.reshape(1, d)
    bp1r = bp1.reshape(1, -1)
    bp2r = bp2.reshape(1, 1)

    deg_parts = _sc_degree(dst, ew, n_pad)
    parts_t = deg_parts.T

    y1, dinv2 = _tc_prep(xp, W1, parts_t)
    p1 = _sc_aggregate(src, dst, ew, y1, n_pad)
    t1, s1 = _tc_post(p1, y1, dinv2, b1r, n)
    y2 = _tc_bn_mm(t1, s1, g1r, be1r, W2, dinv2, n)
    p2 = _sc_aggregate(src, dst, ew, y2, n_pad)
    t2, s2 = _tc_post(p2, y2, dinv2, b2r, n)
    out = _tc_head(t2, s2, g2r, be2r, batch2d, Wp1, bp1r, Wp2, bp2r,
                   n, n_graphs)
    return out
